# baseline (speedup 1.0000x reference)
import functools

import jax
import jax.numpy as jnp
from jax import lax
from jax.experimental import pallas as pl
from jax.experimental.pallas import tpu as pltpu
from jax.experimental.pallas import tpu_sc as plsc

THEA = 0.5
DELTA = 1.5
MIN_PIXELS = 20.0
R = 16
C = 32
NB = 4
N_PIX = 512 * 512
P = 131072
K = N_PIX // P
HR = P // 512

NW = 32
LPW = NB * N_PIX // NW
VECS = LPW // 16
UNROLL = 32


def _sc_counts(target):
    mesh = plsc.VectorSubcoreMesh(core_axis_name="c", subcore_axis_name="s")

    @functools.partial(
        pl.kernel,
        out_type=jax.ShapeDtypeStruct((NW, R), jnp.float32),
        mesh=mesh,
        compiler_params=pltpu.CompilerParams(needs_layout_passes=False),
        scratch_types=[
            pltpu.VMEM((LPW,), jnp.int32),
            pltpu.VMEM((R,), jnp.float32),
        ],
    )
    def hist_kernel(labs_hbm, out_hbm, buf, hist):
        wid = lax.axis_index("s") * 2 + lax.axis_index("c")
        base = wid * LPW
        pltpu.sync_copy(labs_hbm.at[pl.ds(base, LPW)], buf)
        hist[...] = jnp.zeros((R,), jnp.float32)
        ones = jnp.ones((16,), jnp.float32)

        def body(i, carry):
            for u in range(UNROLL):
                idx = buf[pl.ds((i * UNROLL + u) * 16, 16)]
                plsc.addupdate_scatter(hist, [idx], ones)
            return carry

        lax.fori_loop(0, VECS // UNROLL, body, 0)
        pltpu.sync_copy(hist, out_hbm.at[wid])

    return hist_kernel(target.reshape(NB * N_PIX))


def _onehot(lab_ref):
    labs = lab_ref[0]
    iota = jax.lax.broadcasted_iota(jnp.int32, (R, P), 0)
    return jnp.where(labs == iota, jnp.float32(1), jnp.float32(0))


def _pass1_body(pred_ref, lab_ref, sums_ref):
    k = pl.program_id(1)
    feat = pred_ref[0].reshape(C, P)
    oh = _onehot(lab_ref)
    psums = jax.lax.dot_general(
        feat, oh, (((1,), (1,)), ((), ())),
        preferred_element_type=jnp.float32)

    @pl.when(k == 0)
    def _():
        sums_ref[...] = psums[None]

    @pl.when(k != 0)
    def _():
        sums_ref[...] += psums[None]


def _pass2_body(pred_ref, lab_ref, sums_ref, cnts_ref, rsq_ref, out_ref,
                means_ref):
    b = pl.program_id(0)
    k = pl.program_id(1)
    feat = pred_ref[0].reshape(C, P)
    oh = _onehot(lab_ref)

    @pl.when(k == 0)
    def _():
        cnt_b = jnp.sum(cnts_ref[pl.ds(b * 8, 8)], axis=0,
                        keepdims=True)
        safe = jnp.maximum(cnt_b, 1.0)
        means_ref[...] = sums_ref[pl.ds(b, 1)][0] / safe

    meanpx = jax.lax.dot_general(
        means_ref[...], oh, (((1,), (0,)), ((), ())),
        preferred_element_type=jnp.float32)
    diff = feat - meanpx
    dsq = jax.lax.dot_general(
        jnp.ones((1, C), jnp.float32), diff * diff,
        (((1,), (0,)), ((), ())),
        preferred_element_type=jnp.float32)
    d = jnp.sqrt(dsq)
    r = jnp.maximum(d - THEA, 0.0)
    r2 = r * r
    prsq = jax.lax.dot_general(
        r2, oh, (((1,), (1,)), ((), ())),
        preferred_element_type=jnp.float32)

    @pl.when(k == 0)
    def _():
        rsq_ref[pl.ds(b, 1)] = prsq[None]

    @pl.when(k != 0)
    def _():
        rsq_ref[pl.ds(b, 1)] += prsq[None]

    @pl.when((b == NB - 1) & (k == K - 1))
    def _():
        total = jnp.float32(0.0)
        for bb in range(NB):
            cnts2 = jnp.sum(cnts_ref[pl.ds(bb * 8, 8)], axis=0,
                            keepdims=True)
            sums2 = sums_ref[bb]
            rsq2 = rsq_ref[bb]
            valid = (cnts2 > MIN_PIXELS).astype(jnp.float32)
            safe_c = jnp.maximum(cnts2, 1.0)
            means = sums2 / safe_c
            n_valid = jnp.maximum(jnp.sum(valid), 1.0)
            loss_var = jnp.sum(valid * (rsq2 / safe_c)) / n_valid
            diffp = means[:, :, None] - means[:, None, :]
            psq = jnp.sum(diffp * diffp, axis=0)
            pdist = jnp.sqrt(psq + 1e-12)
            eye = (jax.lax.broadcasted_iota(jnp.int32, (R, R), 0) ==
                   jax.lax.broadcasted_iota(jnp.int32, (R, R), 1))
            pm = (valid * valid[0][:, None]) * (1.0 - eye.astype(jnp.float32))
            rdis = jnp.maximum(2.0 * DELTA - pdist, 0.0)
            cntp = jnp.maximum(jnp.sum(pm), 1.0)
            loss_dis = jnp.sum(pm * rdis * rdis) / cntp
            mnorm = jnp.sqrt(jnp.sum(means * means, axis=0,
                                     keepdims=True))
            loss_reg = jnp.sum(valid * mnorm) / n_valid
            total = total + (loss_var + loss_dis + 0.001 * loss_reg)
        out_ref[...] = jnp.broadcast_to(total / NB, (1, 1))


def kernel(predict, target):
    cnts = _sc_counts(target)
    labs = target.reshape(NB * K, 1, P)
    sums = pl.pallas_call(
        _pass1_body,
        grid=(NB, K),
        in_specs=[
            pl.BlockSpec((1, C, HR, 512), lambda b, k: (b, 0, k, 0)),
            pl.BlockSpec((1, 1, P), lambda b, k: (b * K + k, 0, 0)),
        ],
        out_specs=pl.BlockSpec((1, C, R), lambda b, k: (b, 0, 0)),
        out_shape=jax.ShapeDtypeStruct((NB, C, R), jnp.float32),
    )(predict, labs)
    outs = pl.pallas_call(
        _pass2_body,
        grid=(NB, K),
        in_specs=[
            pl.BlockSpec((1, C, HR, 512), lambda b, k: (b, 0, k, 0)),
            pl.BlockSpec((1, 1, P), lambda b, k: (b * K + k, 0, 0)),
            pl.BlockSpec((NB, C, R), lambda b, k: (0, 0, 0)),
            pl.BlockSpec((NW, R), lambda b, k: (0, 0)),
        ],
        out_specs=[
            pl.BlockSpec((NB, 1, R), lambda b, k: (0, 0, 0)),
            pl.BlockSpec((1, 1), lambda b, k: (0, 0)),
        ],
        out_shape=[
            jax.ShapeDtypeStruct((NB, 1, R), jnp.float32),
            jax.ShapeDtypeStruct((1, 1), jnp.float32),
        ],
        scratch_shapes=[pltpu.VMEM((C, R), jnp.float32)],
    )(predict, labs, sums, cnts)
    return outs[1][0, 0]
